# baseline (speedup 1.0000x reference)
import functools

import jax
import jax.numpy as jnp
from jax import lax
from jax.experimental import pallas as pl
from jax.experimental.pallas import tpu as pltpu
from jax.experimental.pallas import tpu_sc as plsc

_TOP_P = 0.8
_E = 64
_BLK_T = 1024
_NW = 32
_CHUNK = 512


def _mm_softmax_body(x_ref, wt_ref, p_ref):
    x = x_ref[...]
    wt = wt_ref[...]
    logits = jax.lax.dot_general(
        x, wt, (((1,), (0,)), ((), ())),
        preferred_element_type=jnp.float32,
        precision=jax.lax.Precision.DEFAULT,
    )
    m = jnp.max(logits, axis=1, keepdims=True)
    ex = jnp.exp(logits - m)
    p_ref[...] = ex / jnp.sum(ex, axis=1, keepdims=True)


def _vsort(x):
    return plsc.sort_key_val(x, x)[0]


def _merge16(a, b):
    rb = lax.rev(b, (0,))
    lo = jnp.minimum(a, rb)
    hi = jnp.maximum(a, rb)
    return _vsort(lo), _vsort(hi)


def _gate_row(ibuf, obuf, r):
    v = [ibuf[pl.ds(64 * r + 16 * k, 16)] for k in range(4)]
    s4 = [_vsort(vk) for vk in v]
    a0, a1 = _merge16(s4[0], s4[1])
    b0, b1 = _merge16(s4[2], s4[3])
    rb0 = lax.rev(b1, (0,))
    rb1 = lax.rev(b0, (0,))
    l0 = jnp.minimum(a0, rb0)
    h0 = jnp.maximum(a0, rb0)
    l1 = jnp.minimum(a1, rb1)
    h1 = jnp.maximum(a1, rb1)
    s = [_vsort(jnp.minimum(l0, l1)), _vsort(jnp.maximum(l0, l1)),
         _vsort(jnp.minimum(h0, h1)), _vsort(jnp.maximum(h0, h1))]
    c = [plsc.cumsum(si) for si in s]
    t = [ci[15] for ci in c]
    h3 = t[3]
    h2 = h3 + t[2]
    h1s = h2 + t[1]
    h0s = h1s + t[0]
    big = jnp.float32(3.4e38)
    tau_v = jnp.full((16,), big, jnp.float32)
    for hi, ci, si in zip((h0s, h1s, h2, h3), c, s):
        g = hi - ci
        tau_v = jnp.minimum(tau_v, jnp.where(g <= _TOP_P, si, big))
    tau = jnp.min(tau_v)
    for k in range(4):
        vk = v[k]
        score = (1.0 + vk) - vk
        obuf[pl.ds(64 * r + 16 * k, 16)] = jnp.where(vk >= tau, score, 0.0)


def _make_sc_gate(n_tok):
    rows_w = n_tok // _NW
    mesh = plsc.VectorSubcoreMesh(core_axis_name="c", subcore_axis_name="s")

    @functools.partial(
        pl.kernel,
        mesh=mesh,
        out_type=jax.ShapeDtypeStruct((n_tok * _E,), jnp.float32),
        scratch_types=[
            pltpu.VMEM((_CHUNK * _E,), jnp.float32),
            pltpu.VMEM((_CHUNK * _E,), jnp.float32),
        ],
        compiler_params=pltpu.CompilerParams(needs_layout_passes=False),
    )
    def sc_gate(probs_hbm, out_hbm, ibuf, obuf):
        wid = lax.axis_index("s") * 2 + lax.axis_index("c")
        base = wid * rows_w

        def do_chunk(ci, _):
            cb = (base + ci * _CHUNK) * _E
            pltpu.sync_copy(probs_hbm.at[pl.ds(cb, _CHUNK * _E)], ibuf)

            @plsc.parallel_loop(0, _CHUNK, 1, unroll=2)
            def row_fn(r):
                _gate_row(ibuf, obuf, r)
            pltpu.sync_copy(obuf, out_hbm.at[pl.ds(cb, _CHUNK * _E)])
            return 0

        lax.fori_loop(0, rows_w // _CHUNK, do_chunk, 0)

    return sc_gate


def _mm_softmax(x, wt):
    n_tok, hidden = x.shape
    return pl.pallas_call(
        _mm_softmax_body,
        grid=(n_tok // _BLK_T,),
        in_specs=[
            pl.BlockSpec((_BLK_T, hidden), lambda i: (i, 0)),
            pl.BlockSpec((hidden, _E), lambda i: (0, 0)),
        ],
        out_specs=pl.BlockSpec((_BLK_T, _E), lambda i: (i, 0)),
        out_shape=jax.ShapeDtypeStruct((n_tok, _E), jnp.float32),
    )(x, wt)


def kernel(routing_inputs, W):
    n_tok, hidden = routing_inputs.shape
    wt = W.T
    half = n_tok // 2
    gate = _make_sc_gate(half)
    probs1 = _mm_softmax(routing_inputs[:half], wt)
    probs2 = _mm_softmax(routing_inputs[half:], wt)
    out1 = gate(probs1.reshape(-1))
    out2 = gate(probs2.reshape(-1))
    return jnp.concatenate(
        [out1.reshape(half, _E), out2.reshape(half, _E)], axis=0)


# speedup vs baseline: 2.2962x; 2.2962x over previous
import functools

import jax
import jax.numpy as jnp
from jax import lax
from jax.experimental import pallas as pl
from jax.experimental.pallas import tpu as pltpu
from jax.experimental.pallas import tpu_sc as plsc

_TOP_P = 0.8
_E = 64
_BLK_T = 1024
_NW = 32
_CHUNK = 512


def _mm_softmax_body(x_ref, wt_ref, p_ref):
    x = x_ref[...]
    wt = wt_ref[...]
    logits = jax.lax.dot_general(
        x, wt, (((1,), (0,)), ((), ())),
        preferred_element_type=jnp.float32,
        precision=jax.lax.Precision.DEFAULT,
    )
    m = jnp.max(logits, axis=1, keepdims=True)
    ex = jnp.exp(logits - m)
    p_ref[...] = ex / jnp.sum(ex, axis=1, keepdims=True)


def _vsort(x):
    return plsc.sort_key_val(x, x)[0]


def _merge16(a, b):
    rb = lax.rev(b, (0,))
    lo = jnp.minimum(a, rb)
    hi = jnp.maximum(a, rb)
    return _vsort(lo), _vsort(hi)


def _gate_row(ibuf, obuf, r):
    v = [ibuf[pl.ds(64 * r + 16 * k, 16)] for k in range(4)]
    s4 = [_vsort(vk) for vk in v]
    a0, a1 = _merge16(s4[0], s4[1])
    b0, b1 = _merge16(s4[2], s4[3])
    rb0 = lax.rev(b1, (0,))
    rb1 = lax.rev(b0, (0,))
    l0 = jnp.minimum(a0, rb0)
    h0 = jnp.maximum(a0, rb0)
    l1 = jnp.minimum(a1, rb1)
    h1 = jnp.maximum(a1, rb1)
    s = [_vsort(jnp.minimum(l0, l1)), _vsort(jnp.maximum(l0, l1)),
         _vsort(jnp.minimum(h0, h1)), _vsort(jnp.maximum(h0, h1))]
    c = [plsc.cumsum(si) for si in s]
    t = [ci[15] for ci in c]
    h3 = t[3]
    h2 = h3 + t[2]
    h1s = h2 + t[1]
    h0s = h1s + t[0]
    big = jnp.float32(3.4e38)
    tau_v = jnp.full((16,), big, jnp.float32)
    for hi, ci, si in zip((h0s, h1s, h2, h3), c, s):
        g = hi - ci
        tau_v = jnp.minimum(tau_v, jnp.where(g <= _TOP_P, si, big))
    tau = jnp.min(tau_v)
    for k in range(4):
        vk = v[k]
        score = (1.0 + vk) - vk
        obuf[pl.ds(64 * r + 16 * k, 16)] = jnp.where(vk >= tau, score, 0.0)


def _make_sc_gate(n_tok):
    rows_w = n_tok // _NW
    mesh = plsc.VectorSubcoreMesh(core_axis_name="c", subcore_axis_name="s")

    @functools.partial(
        pl.kernel,
        mesh=mesh,
        out_type=jax.ShapeDtypeStruct((n_tok * _E,), jnp.float32),
        scratch_types=[
            pltpu.VMEM((_CHUNK * _E,), jnp.float32),
            pltpu.VMEM((_CHUNK * _E,), jnp.float32),
        ],
        compiler_params=pltpu.CompilerParams(needs_layout_passes=False),
    )
    def sc_gate(probs_hbm, out_hbm, ibuf, obuf):
        wid = lax.axis_index("s") * 2 + lax.axis_index("c")
        base = wid * rows_w

        def do_chunk(ci, _):
            cb = (base + ci * _CHUNK) * _E
            pltpu.sync_copy(probs_hbm.at[pl.ds(cb, _CHUNK * _E)], ibuf)

            @plsc.parallel_loop(0, _CHUNK, 1, unroll=2)
            def row_fn(r):
                _gate_row(ibuf, obuf, r)
            pltpu.sync_copy(obuf, out_hbm.at[pl.ds(cb, _CHUNK * _E)])
            return 0

        lax.fori_loop(0, rows_w // _CHUNK, do_chunk, 0)

    return sc_gate


def _mm_softmax(x, wt, off_blocks, out_tok):
    hidden = x.shape[1]
    return pl.pallas_call(
        _mm_softmax_body,
        grid=(out_tok // _BLK_T,),
        in_specs=[
            pl.BlockSpec((_BLK_T, hidden), lambda i: (i + off_blocks, 0)),
            pl.BlockSpec((hidden, _E), lambda i: (0, 0)),
        ],
        out_specs=pl.BlockSpec((_BLK_T, _E), lambda i: (i, 0)),
        out_shape=jax.ShapeDtypeStruct((out_tok, _E), jnp.float32),
    )(x, wt)


def kernel(routing_inputs, W):
    n_tok, hidden = routing_inputs.shape
    wt = W.T
    half = n_tok // 2
    gate = _make_sc_gate(half)
    probs1 = _mm_softmax(routing_inputs, wt, 0, half)
    probs2 = _mm_softmax(routing_inputs, wt, half // _BLK_T, half)
    out1 = gate(probs1.reshape(-1))
    out2 = gate(probs2.reshape(-1))
    return jnp.concatenate(
        [out1.reshape(half, _E), out2.reshape(half, _E)], axis=0)


# speedup vs baseline: 2.5983x; 1.1316x over previous
import functools

import jax
import jax.numpy as jnp
from jax import lax
from jax.experimental import pallas as pl
from jax.experimental.pallas import tpu as pltpu
from jax.experimental.pallas import tpu_sc as plsc

_TOP_P = 0.8
_E = 64
_BLK_T = 1024
_NW = 32
_CHUNK = 512


def _mm_softmax_body(x_ref, wt_ref, p_ref):
    x = x_ref[...]
    wt = wt_ref[...]
    logits = jax.lax.dot_general(
        x, wt, (((1,), (0,)), ((), ())),
        preferred_element_type=jnp.float32,
        precision=jax.lax.Precision.DEFAULT,
    )
    m = jnp.max(logits, axis=1, keepdims=True)
    ex = jnp.exp(logits - m)
    p_ref[...] = ex / jnp.sum(ex, axis=1, keepdims=True)


def _vsort(x):
    return plsc.sort_key_val(x, x)[0]


def _merge16(a, b):
    rb = lax.rev(b, (0,))
    lo = jnp.minimum(a, rb)
    hi = jnp.maximum(a, rb)
    return _vsort(lo), _vsort(hi)


def _gate_row(ibuf, obuf, r):
    v = [ibuf[r, pl.ds(16 * k, 16)] for k in range(4)]
    s4 = [_vsort(vk) for vk in v]
    a0, a1 = _merge16(s4[0], s4[1])
    b0, b1 = _merge16(s4[2], s4[3])
    rb0 = lax.rev(b1, (0,))
    rb1 = lax.rev(b0, (0,))
    l0 = jnp.minimum(a0, rb0)
    h0 = jnp.maximum(a0, rb0)
    l1 = jnp.minimum(a1, rb1)
    h1 = jnp.maximum(a1, rb1)
    s = [_vsort(jnp.minimum(l0, l1)), _vsort(jnp.maximum(l0, l1)),
         _vsort(jnp.minimum(h0, h1)), _vsort(jnp.maximum(h0, h1))]
    c = [plsc.cumsum(si) for si in s]
    t = [ci[15] for ci in c]
    h3 = t[3]
    h2 = h3 + t[2]
    h1s = h2 + t[1]
    h0s = h1s + t[0]
    big = jnp.float32(3.4e38)
    tau_v = jnp.full((16,), big, jnp.float32)
    for hi, ci, si in zip((h0s, h1s, h2, h3), c, s):
        g = hi - ci
        tau_v = jnp.minimum(tau_v, jnp.where(g <= _TOP_P, si, big))
    tau = jnp.min(tau_v)
    for k in range(4):
        vk = v[k]
        score = (1.0 + vk) - vk
        obuf[r, pl.ds(16 * k, 16)] = jnp.where(vk >= tau, score, 0.0)


def _make_sc_gate(n_tok):
    rows_w = n_tok // _NW
    mesh = plsc.VectorSubcoreMesh(core_axis_name="c", subcore_axis_name="s")

    @functools.partial(
        pl.kernel,
        mesh=mesh,
        out_type=jax.ShapeDtypeStruct((n_tok, _E), jnp.float32),
        scratch_types=[
            pltpu.VMEM((_CHUNK, _E), jnp.float32),
            pltpu.VMEM((_CHUNK, _E), jnp.float32),
        ],
        compiler_params=pltpu.CompilerParams(needs_layout_passes=False),
    )
    def sc_gate(probs_hbm, out_hbm, ibuf, obuf):
        wid = lax.axis_index("s") * 2 + lax.axis_index("c")
        base = wid * rows_w

        def do_chunk(ci, _):
            cb = base + ci * _CHUNK
            pltpu.sync_copy(probs_hbm.at[pl.ds(cb, _CHUNK)], ibuf)

            def row_fn(r, _c):
                _gate_row(ibuf, obuf, r)
                return 0

            lax.fori_loop(0, _CHUNK, row_fn, 0)
            pltpu.sync_copy(obuf, out_hbm.at[pl.ds(cb, _CHUNK)])
            return 0

        lax.fori_loop(0, rows_w // _CHUNK, do_chunk, 0)

    return sc_gate


def kernel(routing_inputs, W):
    n_tok, hidden = routing_inputs.shape
    wt = W.T
    probs = pl.pallas_call(
        _mm_softmax_body,
        grid=(n_tok // _BLK_T,),
        in_specs=[
            pl.BlockSpec((_BLK_T, hidden), lambda i: (i, 0)),
            pl.BlockSpec((hidden, _E), lambda i: (0, 0)),
        ],
        out_specs=pl.BlockSpec((_BLK_T, _E), lambda i: (i, 0)),
        out_shape=jax.ShapeDtypeStruct((n_tok, _E), jnp.float32),
    )(routing_inputs, wt)
    return _make_sc_gate(n_tok)(probs)


# speedup vs baseline: 2.6408x; 1.0163x over previous
import functools

import jax
import jax.numpy as jnp
from jax import lax
from jax.experimental import pallas as pl
from jax.experimental.pallas import tpu as pltpu
from jax.experimental.pallas import tpu_sc as plsc

_TOP_P = 0.8
_E = 64
_BLK_T = 1024
_NW = 32
_CHUNK = 256


def _mm_softmax_body(x_ref, wt_ref, p_ref):
    x = x_ref[...]
    wt = wt_ref[...]
    logits = jax.lax.dot_general(
        x, wt, (((1,), (0,)), ((), ())),
        preferred_element_type=jnp.float32,
        precision=jax.lax.Precision.DEFAULT,
    )
    m = jnp.max(logits, axis=1, keepdims=True)
    ex = jnp.exp(logits - m)
    p_ref[...] = ex / jnp.sum(ex, axis=1, keepdims=True)


def _vsort(x):
    return plsc.sort_key_val(x, x)[0]


def _merge16(a, b):
    rb = lax.rev(b, (0,))
    lo = jnp.minimum(a, rb)
    hi = jnp.maximum(a, rb)
    return _vsort(lo), _vsort(hi)


def _gate_row(ibuf, obuf, r):
    v = [ibuf[r, pl.ds(16 * k, 16)] for k in range(4)]
    s4 = [_vsort(vk) for vk in v]
    a0, a1 = _merge16(s4[0], s4[1])
    b0, b1 = _merge16(s4[2], s4[3])
    rb0 = lax.rev(b1, (0,))
    rb1 = lax.rev(b0, (0,))
    l0 = jnp.minimum(a0, rb0)
    h0 = jnp.maximum(a0, rb0)
    l1 = jnp.minimum(a1, rb1)
    h1 = jnp.maximum(a1, rb1)
    s = [_vsort(jnp.minimum(l0, l1)), _vsort(jnp.maximum(l0, l1)),
         _vsort(jnp.minimum(h0, h1)), _vsort(jnp.maximum(h0, h1))]
    c = [plsc.cumsum(si) for si in s]
    t = [ci[15] for ci in c]
    h3 = t[3]
    h2 = h3 + t[2]
    h1s = h2 + t[1]
    h0s = h1s + t[0]
    big = jnp.float32(3.4e38)
    tau_v = jnp.full((16,), big, jnp.float32)
    for hi, ci, si in zip((h0s, h1s, h2, h3), c, s):
        g = hi - ci
        tau_v = jnp.minimum(tau_v, jnp.where(g <= _TOP_P, si, big))
    tau = jnp.min(tau_v)
    for k in range(4):
        vk = v[k]
        score = (1.0 + vk) - vk
        obuf[r, pl.ds(16 * k, 16)] = jnp.where(vk >= tau, score, 0.0)


def _make_sc_gate(n_tok):
    rows_w = n_tok // _NW
    mesh = plsc.VectorSubcoreMesh(core_axis_name="c", subcore_axis_name="s")

    nchunks = rows_w // _CHUNK

    @functools.partial(
        pl.kernel,
        mesh=mesh,
        out_type=jax.ShapeDtypeStruct((n_tok, _E), jnp.float32),
        scratch_types=[
            pltpu.VMEM((_CHUNK, _E), jnp.float32),
            pltpu.VMEM((_CHUNK, _E), jnp.float32),
            pltpu.VMEM((_CHUNK, _E), jnp.float32),
            pltpu.VMEM((_CHUNK, _E), jnp.float32),
            pltpu.SemaphoreType.DMA,
            pltpu.SemaphoreType.DMA,
            pltpu.SemaphoreType.DMA,
            pltpu.SemaphoreType.DMA,
        ],
        compiler_params=pltpu.CompilerParams(needs_layout_passes=False),
    )
    def sc_gate(probs_hbm, out_hbm, ib0, ib1, ob0, ob1, si0, si1, so0, so1):
        wid = lax.axis_index("s") * 2 + lax.axis_index("c")
        base = wid * rows_w
        ibs, obs, sis, sos = (ib0, ib1), (ob0, ob1), (si0, si1), (so0, so1)

        in_copies = {}
        out_copies = {}
        in_copies[0] = pltpu.async_copy(
            probs_hbm.at[pl.ds(base, _CHUNK)], ibs[0], sis[0])
        for c in range(nchunks):
            ib, ob = ibs[c % 2], obs[c % 2]
            in_copies[c].wait()
            if c + 1 < nchunks:
                in_copies[c + 1] = pltpu.async_copy(
                    probs_hbm.at[pl.ds(base + (c + 1) * _CHUNK, _CHUNK)],
                    ibs[(c + 1) % 2], sis[(c + 1) % 2])
            if c >= 2:
                out_copies[c - 2].wait()

            def row_fn(r, _c):
                _gate_row(ib, ob, r)
                return 0

            lax.fori_loop(0, _CHUNK, row_fn, 0)
            out_copies[c] = pltpu.async_copy(
                ob, out_hbm.at[pl.ds(base + c * _CHUNK, _CHUNK)], sos[c % 2])
        out_copies[nchunks - 2].wait()
        out_copies[nchunks - 1].wait()

    return sc_gate


def kernel(routing_inputs, W):
    n_tok, hidden = routing_inputs.shape
    wt = W.T
    probs = pl.pallas_call(
        _mm_softmax_body,
        grid=(n_tok // _BLK_T,),
        in_specs=[
            pl.BlockSpec((_BLK_T, hidden), lambda i: (i, 0)),
            pl.BlockSpec((hidden, _E), lambda i: (0, 0)),
        ],
        out_specs=pl.BlockSpec((_BLK_T, _E), lambda i: (i, 0)),
        out_shape=jax.ShapeDtypeStruct((n_tok, _E), jnp.float32),
    )(routing_inputs, wt)
    return _make_sc_gate(n_tok)(probs)


# speedup vs baseline: 2.6880x; 1.0179x over previous
import functools

import jax
import jax.numpy as jnp
from jax import lax
from jax.experimental import pallas as pl
from jax.experimental.pallas import tpu as pltpu
from jax.experimental.pallas import tpu_sc as plsc

_TOP_P = 0.8
_E = 64
_BLK_T = 1024
_NW = 32
_CHUNK = 128


def _mm_softmax_body(x_ref, wt_ref, p_ref):
    x = x_ref[...]
    wt = wt_ref[...]
    logits = jax.lax.dot_general(
        x, wt, (((1,), (0,)), ((), ())),
        preferred_element_type=jnp.float32,
        precision=jax.lax.Precision.DEFAULT,
    )
    m = jnp.max(logits, axis=1, keepdims=True)
    ex = jnp.exp(logits - m)
    p_ref[...] = ex / jnp.sum(ex, axis=1, keepdims=True)


def _vsort(x):
    return plsc.sort_key_val(x, x)[0]


def _merge16(a, b):
    rb = lax.rev(b, (0,))
    lo = jnp.minimum(a, rb)
    hi = jnp.maximum(a, rb)
    return _vsort(lo), _vsort(hi)


def _gate_row(ibuf, obuf, r):
    v = [ibuf[r, pl.ds(16 * k, 16)] for k in range(4)]
    s4 = [_vsort(vk) for vk in v]
    a0, a1 = _merge16(s4[0], s4[1])
    b0, b1 = _merge16(s4[2], s4[3])
    rb0 = lax.rev(b1, (0,))
    rb1 = lax.rev(b0, (0,))
    l0 = jnp.minimum(a0, rb0)
    h0 = jnp.maximum(a0, rb0)
    l1 = jnp.minimum(a1, rb1)
    h1 = jnp.maximum(a1, rb1)
    s = [_vsort(jnp.minimum(l0, l1)), _vsort(jnp.maximum(l0, l1)),
         _vsort(jnp.minimum(h0, h1)), _vsort(jnp.maximum(h0, h1))]
    c = [plsc.cumsum(si) for si in s]
    t = [ci[15] for ci in c]
    h3 = t[3]
    h2 = h3 + t[2]
    h1s = h2 + t[1]
    h0s = h1s + t[0]
    big = jnp.float32(3.4e38)
    tau_v = jnp.full((16,), big, jnp.float32)
    for hi, ci, si in zip((h0s, h1s, h2, h3), c, s):
        g = hi - ci
        tau_v = jnp.minimum(tau_v, jnp.where(g <= _TOP_P, si, big))
    tau = jnp.min(tau_v)
    for k in range(4):
        vk = v[k]
        score = (1.0 + vk) - vk
        obuf[r, pl.ds(16 * k, 16)] = jnp.where(vk >= tau, score, 0.0)


def _make_sc_gate(n_tok):
    rows_w = n_tok // _NW
    mesh = plsc.VectorSubcoreMesh(core_axis_name="c", subcore_axis_name="s")

    nchunks = rows_w // _CHUNK

    @functools.partial(
        pl.kernel,
        mesh=mesh,
        out_type=jax.ShapeDtypeStruct((n_tok, _E), jnp.float32),
        scratch_types=[
            pltpu.VMEM((_CHUNK, _E), jnp.float32),
            pltpu.VMEM((_CHUNK, _E), jnp.float32),
            pltpu.VMEM((_CHUNK, _E), jnp.float32),
            pltpu.VMEM((_CHUNK, _E), jnp.float32),
            pltpu.SemaphoreType.DMA,
            pltpu.SemaphoreType.DMA,
            pltpu.SemaphoreType.DMA,
            pltpu.SemaphoreType.DMA,
        ],
        compiler_params=pltpu.CompilerParams(needs_layout_passes=False),
    )
    def sc_gate(probs_hbm, out_hbm, ib0, ib1, ob0, ob1, si0, si1, so0, so1):
        wid = lax.axis_index("s") * 2 + lax.axis_index("c")
        base = wid * rows_w
        ibs, obs, sis, sos = (ib0, ib1), (ob0, ob1), (si0, si1), (so0, so1)

        in_copies = {}
        out_copies = {}
        in_copies[0] = pltpu.async_copy(
            probs_hbm.at[pl.ds(base, _CHUNK)], ibs[0], sis[0])
        for c in range(nchunks):
            ib, ob = ibs[c % 2], obs[c % 2]
            in_copies[c].wait()
            if c + 1 < nchunks:
                in_copies[c + 1] = pltpu.async_copy(
                    probs_hbm.at[pl.ds(base + (c + 1) * _CHUNK, _CHUNK)],
                    ibs[(c + 1) % 2], sis[(c + 1) % 2])
            if c >= 2:
                out_copies[c - 2].wait()

            def row_fn(r, _c):
                _gate_row(ib, ob, r)
                return 0

            lax.fori_loop(0, _CHUNK, row_fn, 0)
            out_copies[c] = pltpu.async_copy(
                ob, out_hbm.at[pl.ds(base + c * _CHUNK, _CHUNK)], sos[c % 2])
        out_copies[nchunks - 2].wait()
        out_copies[nchunks - 1].wait()

    return sc_gate


def kernel(routing_inputs, W):
    n_tok, hidden = routing_inputs.shape
    wt = W.T
    probs = pl.pallas_call(
        _mm_softmax_body,
        grid=(n_tok // _BLK_T,),
        in_specs=[
            pl.BlockSpec((_BLK_T, hidden), lambda i: (i, 0)),
            pl.BlockSpec((hidden, _E), lambda i: (0, 0)),
        ],
        out_specs=pl.BlockSpec((_BLK_T, _E), lambda i: (i, 0)),
        out_shape=jax.ShapeDtypeStruct((n_tok, _E), jnp.float32),
    )(routing_inputs, wt)
    return _make_sc_gate(n_tok)(probs)
